# baseline (speedup 1.0000x reference)
import functools

import jax
import jax.numpy as jnp
from jax import lax
from jax.experimental import pallas as pl
from jax.experimental.pallas import tpu as pltpu
from jax.experimental.pallas import tpu_sc as plsc

_B, _T, _H, _W, _C = 16, 16, 14, 14, 768
_NSLAB = _B * _T * _H
_NW = 32
_NGRP = _B * _H
_GPW = _NGRP // _NW
_HC = _C // 2


@functools.cache
def _build_sc_patch_shift():
    @functools.partial(
        pl.kernel,
        mesh=plsc.VectorSubcoreMesh(core_axis_name="c", subcore_axis_name="s"),
        out_type=jax.ShapeDtypeStruct((_NSLAB, _W, _C), jnp.float32),
        scratch_types=[
            pltpu.VMEM((_T, _W, _HC), jnp.float32),
            pltpu.VMEM((2, _W, _HC), jnp.float32),
            pltpu.SemaphoreType.DMA,
            pltpu.SemaphoreType.DMA,
        ],
    )
    def _sc_patch_shift(x_hbm, out_hbm, bank, stage, fsem, wsem):
        wid = lax.axis_index("s") * 2 + lax.axis_index("c")

        def fwait():
            pltpu.make_async_copy(
                x_hbm.at[0, :, pl.ds(0, _HC)], bank.at[0], fsem).wait()

        def wwait():
            pltpu.make_async_copy(
                stage.at[0], out_hbm.at[0, :, pl.ds(0, _HC)], wsem).wait()

        def phase_body(ph, carry):
            gi = lax.div(ph, 2)
            half = ph - gi * 2
            g = wid * _GPW + gi
            b = lax.div(g, _H)
            h = g - b * _H
            c0 = half * _HC
            sbase = b * _T * _H + h

            for i in range(_T):
                ts = (_T - 4 + i) % _T
                pltpu.async_copy(
                    x_hbm.at[sbase + ts * _H, :, pl.ds(c0, _HC)],
                    bank.at[ts], fsem)

            svals = []
            for w in range(_W):
                p = h * _W + w
                h7 = lax.div(p, 7)
                w7 = p - h7 * 7
                code = (w7 % 3) * 3 + (h7 % 3)
                s = jnp.where(code == 0, -4,
                    jnp.where(code == 1, 1,
                    jnp.where(code == 2, 2,
                    jnp.where(code == 3, -1,
                    jnp.where(code == 5, 3,
                    jnp.where(code == 6, -2,
                    jnp.where(code == 7, -3,
                    jnp.where(code == 8, 4,
                        jnp.where(p == 8, 0, -1)))))))))
                svals.append(s)

            def tbody(t, carry2):
                par = t & 1

                @pl.when(t == 0)
                def _():
                    for _i in range(9):
                        fwait()

                @pl.when(jnp.logical_and(t >= 1, t <= 7))
                def _():
                    fwait()

                @pl.when(ph * _T + t >= 2)
                def _():
                    wwait()

                for w in range(_W):
                    src = (t - svals[w] + _T) & (_T - 1)
                    vals = [bank[src, w, pl.ds(j * 16, 16)]
                            for j in range(_HC // 16)]
                    for j, v in enumerate(vals):
                        stage[par, w, pl.ds(j * 16, 16)] = v
                pltpu.async_copy(
                    stage.at[par],
                    out_hbm.at[sbase + t * _H, :, pl.ds(c0, _HC)], wsem)
                return carry2

            lax.fori_loop(0, _T, tbody, 0)
            return carry

        lax.fori_loop(0, 2 * _GPW, phase_body, 0)
        wwait()
        wwait()

    return _sc_patch_shift


def kernel(x):
    x3 = x.reshape(_NSLAB, _W, _C)
    out = _build_sc_patch_shift()(x3)
    return out.reshape(_B, _T, _H, _W, _C)
